# Initial kernel scaffold; baseline (speedup 1.0000x reference)
import functools

import jax
import jax.numpy as jnp
from jax import lax
from jax.experimental import pallas as pl
from jax.experimental.pallas import tpu as pltpu
from jax.experimental.pallas import tpu_sc as plsc

VOCAB = 1000000
EMBED_DIM = 64
RANK = 16


_VCHUNK = 8000


def _fuse_body(a_ref, b_ref, t_ref, o_ref):
    a = a_ref[...]
    b = b_ref[...]
    delta = lax.dot_general(a, b, (((0,), (1,)), ((), ())),
                            preferred_element_type=jnp.float32)
    o_ref[...] = t_ref[...] + delta


def _fuse_table(lora_A, lora_B, table):
    grid = VOCAB // _VCHUNK
    return pl.pallas_call(
        _fuse_body,
        grid=(grid,),
        in_specs=[
            pl.BlockSpec((RANK, _VCHUNK), lambda i: (0, i)),
            pl.BlockSpec((EMBED_DIM, RANK), lambda i: (0, 0)),
            pl.BlockSpec((_VCHUNK, EMBED_DIM), lambda i: (i, 0)),
        ],
        out_specs=pl.BlockSpec((_VCHUNK, EMBED_DIM), lambda i: (i, 0)),
        out_shape=jax.ShapeDtypeStruct((VOCAB, EMBED_DIM), jnp.float32),
    )(lora_A, lora_B, table)


_TCHUNK = 128


def _make_gather(n_tokens):
    info = plsc.get_sparse_core_info()
    nc, ns = info.num_cores, info.num_subcores
    nw = nc * ns
    assert n_tokens % (nw * _TCHUNK) == 0
    per_w = n_tokens // nw
    n_iter = per_w // _TCHUNK
    mesh = plsc.VectorSubcoreMesh(core_axis_name="c", subcore_axis_name="s")

    @functools.partial(
        pl.kernel,
        mesh=mesh,
        out_type=jax.ShapeDtypeStruct((n_tokens, EMBED_DIM), jnp.float32),
        scratch_types=[
            pltpu.VMEM((_TCHUNK,), jnp.int32),
            pltpu.VMEM((_TCHUNK, EMBED_DIM), jnp.float32),
            pltpu.SemaphoreType.DMA,
        ],
    )
    def gather(fused_hbm, idx_hbm, out_hbm, idx_v, rows_v, sem):
        wid = lax.axis_index("s") * nc + lax.axis_index("c")
        base = wid * per_w

        def body(i, carry):
            off = base + i * _TCHUNK
            pltpu.sync_copy(idx_hbm.at[pl.ds(off, _TCHUNK)], idx_v)
            pltpu.async_copy(fused_hbm.at[idx_v], rows_v, sem).wait()
            pltpu.sync_copy(rows_v, out_hbm.at[pl.ds(off, _TCHUNK)])
            return carry

        lax.fori_loop(0, n_iter, body, 0)

    return gather


def kernel(x, table, lora_A, lora_B):
    fused = _fuse_table(lora_A, lora_B, table)
    b, s = x.shape
    x_flat = x.reshape(-1).astype(jnp.int32)
    out = _make_gather(b * s)(fused, x_flat)
    return out.reshape(b, s, EMBED_DIM)


# speedup vs baseline: 7.5110x; 7.5110x over previous
import functools

import jax
import jax.numpy as jnp
from jax import lax
from jax.experimental import pallas as pl
from jax.experimental.pallas import tpu as pltpu
from jax.experimental.pallas import tpu_sc as plsc

VOCAB = 1000000
EMBED_DIM = 64
RANK = 16


_VCHUNK = 8000


def _fuse_body(a_ref, b_ref, t_ref, o_ref):
    a = a_ref[...].reshape(RANK, _VCHUNK)
    b = b_ref[...]
    delta = lax.dot_general(a, b, (((0,), (1,)), ((), ())),
                            preferred_element_type=jnp.float32)
    o_ref[...] = t_ref[...] + delta


def _fuse_table(lora_A, lora_B, table):
    grid = VOCAB // _VCHUNK
    a4 = lora_A.reshape(RANK, grid, 1, _VCHUNK)
    return pl.pallas_call(
        _fuse_body,
        grid=(grid,),
        in_specs=[
            pl.BlockSpec((RANK, 1, 1, _VCHUNK), lambda i: (0, i, 0, 0)),
            pl.BlockSpec((EMBED_DIM, RANK), lambda i: (0, 0)),
            pl.BlockSpec((_VCHUNK, EMBED_DIM), lambda i: (i, 0)),
        ],
        out_specs=pl.BlockSpec((_VCHUNK, EMBED_DIM), lambda i: (i, 0)),
        out_shape=jax.ShapeDtypeStruct((VOCAB, EMBED_DIM), jnp.float32),
    )(a4, lora_B, table)


_TCHUNK = 128


def _make_gather(n_tokens):
    info = plsc.get_sparse_core_info()
    nc, ns = info.num_cores, info.num_subcores
    nw = nc * ns
    assert n_tokens % (nw * _TCHUNK) == 0
    per_w = n_tokens // nw
    n_iter = per_w // _TCHUNK
    mesh = plsc.VectorSubcoreMesh(core_axis_name="c", subcore_axis_name="s")

    @functools.partial(
        pl.kernel,
        mesh=mesh,
        compiler_params=pltpu.CompilerParams(use_tc_tiling_on_sc=False),
        out_type=jax.ShapeDtypeStruct((n_tokens, EMBED_DIM), jnp.float32),
        scratch_types=[
            pltpu.VMEM((_TCHUNK,), jnp.int32),
            pltpu.VMEM((_TCHUNK, EMBED_DIM), jnp.float32),
            pltpu.SemaphoreType.DMA,
        ],
    )
    def gather(fused_hbm, idx_hbm, out_hbm, idx_v, rows_v, sem):
        wid = lax.axis_index("s") * nc + lax.axis_index("c")
        base = wid * per_w

        def body(i, carry):
            off = base + i * _TCHUNK
            pltpu.sync_copy(idx_hbm.at[pl.ds(off, _TCHUNK)], idx_v)
            pltpu.async_copy(fused_hbm.at[idx_v], rows_v, sem).wait()
            pltpu.sync_copy(rows_v, out_hbm.at[pl.ds(off, _TCHUNK)])
            return carry

        lax.fori_loop(0, n_iter, body, 0)

    return gather


def kernel(x, table, lora_A, lora_B):
    fused = _fuse_table(lora_A, lora_B, table)
    b, s = x.shape
    x_flat = x.reshape(-1).astype(jnp.int32)
    out = _make_gather(b * s)(fused, x_flat)
    return out.reshape(b, s, EMBED_DIM)


# speedup vs baseline: 8.3993x; 1.1183x over previous
import functools

import jax
import jax.numpy as jnp
from jax import lax
from jax.experimental import pallas as pl
from jax.experimental.pallas import tpu as pltpu
from jax.experimental.pallas import tpu_sc as plsc

VOCAB = 1000000
EMBED_DIM = 64
RANK = 16


_VCHUNK = 8000


def _fuse_body(a_ref, b_ref, t_ref, o_ref):
    a = a_ref[...].reshape(RANK, _VCHUNK)
    b = b_ref[...]
    delta = lax.dot_general(a, b, (((0,), (1,)), ((), ())),
                            preferred_element_type=jnp.float32)
    o_ref[...] = t_ref[...] + delta


def _fuse_table(lora_A, lora_B, table):
    grid = VOCAB // _VCHUNK
    a4 = lora_A.reshape(RANK, grid, 1, _VCHUNK)
    return pl.pallas_call(
        _fuse_body,
        grid=(grid,),
        in_specs=[
            pl.BlockSpec((RANK, 1, 1, _VCHUNK), lambda i: (0, i, 0, 0)),
            pl.BlockSpec((EMBED_DIM, RANK), lambda i: (0, 0)),
            pl.BlockSpec((_VCHUNK, EMBED_DIM), lambda i: (i, 0)),
        ],
        out_specs=pl.BlockSpec((_VCHUNK, EMBED_DIM), lambda i: (i, 0)),
        out_shape=jax.ShapeDtypeStruct((VOCAB, EMBED_DIM), jnp.float32),
    )(a4, lora_B, table)


_CHUNK = 128
_K = 5
_NB = 2


def _make_gather(b, s):
    n_tokens = b * s
    info = plsc.get_sparse_core_info()
    nc, ns = info.num_cores, info.num_subcores
    nw = nc * ns
    n_rows = n_tokens // _CHUNK
    rows_per_w = n_rows // nw
    n_batch = rows_per_w // _K
    assert n_tokens % (nw * _CHUNK * _K) == 0
    mesh = plsc.VectorSubcoreMesh(core_axis_name="c", subcore_axis_name="s")

    @functools.partial(
        pl.kernel,
        mesh=mesh,
        compiler_params=pltpu.CompilerParams(use_tc_tiling_on_sc=False),
        out_type=jax.ShapeDtypeStruct((n_rows, _CHUNK, EMBED_DIM),
                                      jnp.float32),
        scratch_types=[
            pltpu.VMEM((_NB, _K, _CHUNK), jnp.int32),
            pltpu.VMEM((_NB, _K, _CHUNK, EMBED_DIM), jnp.float32),
            pltpu.SemaphoreType.DMA((_NB,)),
            pltpu.SemaphoreType.DMA,
            pltpu.SemaphoreType.DMA((_NB,)),
        ],
    )
    def gather(fused_hbm, idx_hbm, out_hbm, idx_v, rows_v, isem, gsem, osem):
        fused2 = fused_hbm
        idx2 = idx_hbm
        out3 = out_hbm
        wid = lax.axis_index("s") * nc + lax.axis_index("c")
        row0 = wid * rows_per_w

        for nb in range(_NB):
            pltpu.async_copy(idx2.at[pl.ds(row0 + nb * _K, _K)],
                             idx_v.at[nb], isem.at[nb])

        def body(g, carry):
            nb = lax.rem(g, _NB)
            r = row0 + g * _K
            pltpu.make_async_copy(idx2.at[pl.ds(r, _K)], idx_v.at[nb],
                                  isem.at[nb]).wait()

            @pl.when(g >= _NB)
            def _():
                pltpu.make_async_copy(rows_v.at[nb],
                                      out3.at[pl.ds(r - _NB * _K, _K)],
                                      osem.at[nb]).wait()

            for j in range(_K):
                pltpu.async_copy(fused2.at[idx_v.at[nb, j]],
                                 rows_v.at[nb, j], gsem)
            for j in range(_K):
                pltpu.make_async_copy(fused2.at[idx_v.at[nb, j]],
                                      rows_v.at[nb, j], gsem).wait()

            @pl.when(g + _NB < n_batch)
            def _():
                pltpu.async_copy(idx2.at[pl.ds(r + _NB * _K, _K)],
                                 idx_v.at[nb], isem.at[nb])

            pltpu.async_copy(rows_v.at[nb], out3.at[pl.ds(r, _K)], osem.at[nb])
            return carry

        lax.fori_loop(0, n_batch, body, 0)

        assert n_batch % _NB == 0
        for nb in range(_NB):
            g = n_batch - _NB + nb
            pltpu.make_async_copy(rows_v.at[nb],
                                  out3.at[pl.ds(row0 + g * _K, _K)],
                                  osem.at[nb]).wait()

    return gather


def kernel(x, table, lora_A, lora_B):
    fused = _fuse_table(lora_A, lora_B, table)
    b, s = x.shape
    x2 = x.astype(jnp.int32).reshape(b * s // _CHUNK, _CHUNK)
    out = _make_gather(b, s)(fused, x2)
    return out.reshape(b, s, EMBED_DIM)


# speedup vs baseline: 10.5812x; 1.2598x over previous
import functools

import jax
import jax.numpy as jnp
from jax import lax
from jax.experimental import pallas as pl
from jax.experimental.pallas import tpu as pltpu
from jax.experimental.pallas import tpu_sc as plsc

VOCAB = 1000000
EMBED_DIM = 64
RANK = 16


_PCHUNK = 4000


def _fuse_body(a_lo_ref, a_hi_ref, bt_ref, t_lo_ref, t_hi_ref, o_ref):
    bt = bt_ref[...]
    d_lo = lax.dot_general(a_lo_ref[...].reshape(RANK, _PCHUNK), bt,
                           (((0,), (0,)), ((), ())),
                           preferred_element_type=jnp.float32)
    d_hi = lax.dot_general(a_hi_ref[...].reshape(RANK, _PCHUNK), bt,
                           (((0,), (0,)), ((), ())),
                           preferred_element_type=jnp.float32)
    o_ref[...] = jnp.concatenate(
        [t_lo_ref[...] + d_lo, t_hi_ref[...] + d_hi], axis=1)


def _fuse_table(lora_A, lora_B, table):
    half = VOCAB // 2
    grid = half // _PCHUNK
    a4 = lora_A.reshape(RANK, 2 * grid, 1, _PCHUNK)
    bt = lora_B.T
    return pl.pallas_call(
        _fuse_body,
        grid=(grid,),
        in_specs=[
            pl.BlockSpec((RANK, 1, 1, _PCHUNK), lambda i: (0, i, 0, 0)),
            pl.BlockSpec((RANK, 1, 1, _PCHUNK),
                         lambda i: (0, i + grid, 0, 0)),
            pl.BlockSpec((RANK, EMBED_DIM), lambda i: (0, 0)),
            pl.BlockSpec((_PCHUNK, EMBED_DIM), lambda i: (i, 0)),
            pl.BlockSpec((_PCHUNK, EMBED_DIM), lambda i: (i + grid, 0)),
        ],
        out_specs=pl.BlockSpec((_PCHUNK, 2 * EMBED_DIM), lambda i: (i, 0)),
        out_shape=jax.ShapeDtypeStruct((half, 2 * EMBED_DIM), jnp.float32),
    )(a4, a4, bt, table, table)


_CHUNK = 128
_K = 5
_NB = 2


def _make_gather(b, s):
    n_tokens = b * s
    info = plsc.get_sparse_core_info()
    nc, ns = info.num_cores, info.num_subcores
    nw = nc * ns
    n_rows = n_tokens // _CHUNK
    rows_per_w = n_rows // nw
    n_batch = rows_per_w // _K
    assert n_tokens % (nw * _CHUNK * _K) == 0
    mesh = plsc.VectorSubcoreMesh(core_axis_name="c", subcore_axis_name="s")

    @functools.partial(
        pl.kernel,
        mesh=mesh,
        compiler_params=pltpu.CompilerParams(use_tc_tiling_on_sc=False),
        out_type=jax.ShapeDtypeStruct((n_rows, _CHUNK, EMBED_DIM),
                                      jnp.float32),
        scratch_types=[
            pltpu.VMEM((_NB, _K, _CHUNK), jnp.int32),
            pltpu.VMEM((_NB, _K, _CHUNK, EMBED_DIM), jnp.float32),
            pltpu.SemaphoreType.DMA((_NB,)),
            pltpu.SemaphoreType.DMA,
            pltpu.SemaphoreType.DMA((_NB,)),
        ],
    )
    def gather(fused_hbm, idx_hbm, out_hbm, idx_v, rows_v, isem, gsem, osem):
        fused2 = fused_hbm
        idx2 = idx_hbm
        out3 = out_hbm
        wid = lax.axis_index("s") * nc + lax.axis_index("c")
        row0 = wid * rows_per_w

        for nb in range(_NB):
            pltpu.async_copy(idx2.at[pl.ds(row0 + nb * _K, _K)],
                             idx_v.at[nb], isem.at[nb])

        def body(g, carry):
            nb = lax.rem(g, _NB)
            r = row0 + g * _K
            pltpu.make_async_copy(idx2.at[pl.ds(r, _K)], idx_v.at[nb],
                                  isem.at[nb]).wait()

            for j in range(_K):
                for i in range(_CHUNK // 16):
                    sl = (nb, j, pl.ds(i * 16, 16))
                    v = idx_v[sl]
                    idx_v[sl] = jnp.where(v < VOCAB // 2, 2 * v,
                                          2 * v - (VOCAB - 1))

            @pl.when(g >= _NB)
            def _():
                pltpu.make_async_copy(rows_v.at[nb],
                                      out3.at[pl.ds(r - _NB * _K, _K)],
                                      osem.at[nb]).wait()

            for j in range(_K):
                pltpu.async_copy(fused2.at[idx_v.at[nb, j]],
                                 rows_v.at[nb, j], gsem)
            for j in range(_K):
                pltpu.make_async_copy(fused2.at[idx_v.at[nb, j]],
                                      rows_v.at[nb, j], gsem).wait()

            @pl.when(g + _NB < n_batch)
            def _():
                pltpu.async_copy(idx2.at[pl.ds(r + _NB * _K, _K)],
                                 idx_v.at[nb], isem.at[nb])

            pltpu.async_copy(rows_v.at[nb], out3.at[pl.ds(r, _K)], osem.at[nb])
            return carry

        lax.fori_loop(0, n_batch, body, 0)

        assert n_batch % _NB == 0
        for nb in range(_NB):
            g = n_batch - _NB + nb
            pltpu.make_async_copy(rows_v.at[nb],
                                  out3.at[pl.ds(row0 + g * _K, _K)],
                                  osem.at[nb]).wait()

    return gather


def kernel(x, table, lora_A, lora_B):
    fused = _fuse_table(lora_A, lora_B, table)
    fused2 = fused.reshape(VOCAB, EMBED_DIM)
    b, s = x.shape
    x2 = x.astype(jnp.int32).reshape(b * s // _CHUNK, _CHUNK)
    out = _make_gather(b, s)(fused2, x2)
    return out.reshape(b, s, EMBED_DIM)


# speedup vs baseline: 13.5082x; 1.2766x over previous
import functools

import jax
import jax.numpy as jnp
from jax import lax
from jax.experimental import pallas as pl
from jax.experimental.pallas import tpu as pltpu
from jax.experimental.pallas import tpu_sc as plsc

VOCAB = 1000000
EMBED_DIM = 64
RANK = 16


_PCHUNK = 4000


def _fuse_body(a_lo_ref, a_hi_ref, bt_ref, t_lo_ref, t_hi_ref, o_ref):
    bt = bt_ref[...]
    d_lo = lax.dot_general(a_lo_ref[...].reshape(RANK, _PCHUNK), bt,
                           (((0,), (0,)), ((), ())),
                           preferred_element_type=jnp.float32)
    d_hi = lax.dot_general(a_hi_ref[...].reshape(RANK, _PCHUNK), bt,
                           (((0,), (0,)), ((), ())),
                           preferred_element_type=jnp.float32)
    o_ref[...] = jnp.concatenate(
        [t_lo_ref[...] + d_lo, t_hi_ref[...] + d_hi], axis=1)


def _fuse_table(lora_A, lora_B, table):
    half = VOCAB // 2
    grid = half // _PCHUNK
    a4 = lora_A.reshape(RANK, 2 * grid, 1, _PCHUNK)
    bt = lora_B.T
    return pl.pallas_call(
        _fuse_body,
        grid=(grid,),
        in_specs=[
            pl.BlockSpec((RANK, 1, 1, _PCHUNK), lambda i: (0, i, 0, 0)),
            pl.BlockSpec((RANK, 1, 1, _PCHUNK),
                         lambda i: (0, i + grid, 0, 0)),
            pl.BlockSpec((RANK, EMBED_DIM), lambda i: (0, 0)),
            pl.BlockSpec((_PCHUNK, EMBED_DIM), lambda i: (i, 0)),
            pl.BlockSpec((_PCHUNK, EMBED_DIM), lambda i: (i + grid, 0)),
        ],
        out_specs=pl.BlockSpec((_PCHUNK, 2 * EMBED_DIM), lambda i: (i, 0)),
        out_shape=jax.ShapeDtypeStruct((half, 2 * EMBED_DIM), jnp.float32),
    )(a4, a4, bt, table, table)


_CHUNK = 128
_K = 5
_NB = 2


def _make_gather(b, s):
    n_tokens = b * s
    info = plsc.get_sparse_core_info()
    nc, ns = info.num_cores, info.num_subcores
    nw = nc * ns
    n_rows = n_tokens // _CHUNK
    rows_per_w = n_rows // nw
    n_batch = rows_per_w // _K
    assert n_tokens % (nw * _CHUNK * _K) == 0
    mesh = plsc.VectorSubcoreMesh(core_axis_name="c", subcore_axis_name="s")

    @functools.partial(
        pl.kernel,
        mesh=mesh,
        compiler_params=pltpu.CompilerParams(use_tc_tiling_on_sc=False),
        out_type=jax.ShapeDtypeStruct((n_rows, _CHUNK, EMBED_DIM),
                                      jnp.float32),
        scratch_types=[
            pltpu.VMEM((_NB, _K, _CHUNK), jnp.int32),
            pltpu.VMEM((_NB, _K, _CHUNK, EMBED_DIM), jnp.float32),
            pltpu.SemaphoreType.DMA((_NB,)),
            pltpu.SemaphoreType.DMA,
            pltpu.SemaphoreType.DMA((_NB,)),
        ],
    )
    def gather(fused_hbm, idx_hbm, out_hbm, idx_v, rows_v, isem, gsem, osem):
        fused2 = fused_hbm
        idx2 = idx_hbm
        out3 = out_hbm
        wid = lax.axis_index("s") * nc + lax.axis_index("c")
        row0 = wid * rows_per_w

        for nb in range(_NB):
            pltpu.async_copy(idx2.at[pl.ds(row0 + nb * _K, _K)],
                             idx_v.at[nb], isem.at[nb])

        def body(g, carry):
            nb = lax.rem(g, _NB)
            r = row0 + g * _K
            pltpu.make_async_copy(idx2.at[pl.ds(r, _K)], idx_v.at[nb],
                                  isem.at[nb]).wait()

            for j in range(_K):
                for i in range(_CHUNK // 16):
                    sl = (nb, j, pl.ds(i * 16, 16))
                    v = idx_v[sl]
                    idx_v[sl] = jnp.where(v < VOCAB // 2, 2 * v,
                                          2 * v - (VOCAB - 1))

            @pl.when(g >= _NB)
            def _():
                pltpu.make_async_copy(rows_v.at[nb],
                                      out3.at[pl.ds(r - _NB * _K, _K)],
                                      osem.at[nb]).wait()

            for j in range(_K):
                pltpu.async_copy(fused2.at[idx_v.at[nb, j]],
                                 rows_v.at[nb, j], gsem)
            for j in range(_K):
                pltpu.make_async_copy(fused2.at[idx_v.at[nb, j]],
                                      rows_v.at[nb, j], gsem).wait()

            @pl.when(g + _NB < n_batch)
            def _():
                pltpu.async_copy(idx2.at[pl.ds(r + _NB * _K, _K)],
                                 idx_v.at[nb], isem.at[nb])

            pltpu.async_copy(rows_v.at[nb], out3.at[pl.ds(r, _K)], osem.at[nb])
            return carry

        lax.fori_loop(0, n_batch, body, 0)

        assert n_batch % _NB == 0
        for nb in range(_NB):
            g = n_batch - _NB + nb
            pltpu.make_async_copy(rows_v.at[nb],
                                  out3.at[pl.ds(row0 + g * _K, _K)],
                                  osem.at[nb]).wait()

    return gather


def _epi_body(i_ref, o_ref):
    v = i_ref[...].reshape(128, 100, 128)
    cols = [v[:, m, :].T for m in range(100)]
    o_ref[...] = jnp.concatenate(cols, axis=0)


def _epilogue(out_lin, b, s):
    n = b * s * EMBED_DIM
    rows = s * EMBED_DIM
    grid = b // 128
    flat = out_lin.reshape(n // 128, 128)
    o = pl.pallas_call(
        _epi_body,
        grid=(grid,),
        in_specs=[pl.BlockSpec((rows, 128), lambda i: (i, 0))],
        out_specs=pl.BlockSpec((rows, 128), lambda i: (0, i)),
        out_shape=jax.ShapeDtypeStruct((rows, b), jnp.float32),
    )(flat)
    return o.reshape(s, EMBED_DIM, b).transpose(2, 0, 1)


def kernel(x, table, lora_A, lora_B):
    fused = _fuse_table(lora_A, lora_B, table)
    fused2 = fused.reshape(VOCAB, EMBED_DIM)
    b, s = x.shape
    x2 = x.astype(jnp.int32).reshape(b * s // _CHUNK, _CHUNK)
    out = _make_gather(b, s)(fused2, x2)
    return _epilogue(out, b, s)
